# baseline (speedup 1.0000x reference)
import jax
import jax.numpy as jnp
from jax.experimental import pallas as pl
from jax.experimental.pallas import tpu as pltpu

KNN = 20
NPTS = 1024


def _taylor_cols(x, y, z):
    one = jnp.ones_like(x)
    xx = x * x
    xy = x * y
    xz = x * z
    yy = y * y
    yz = y * z
    zz = z * z
    return [one, x, y, z,
            xx, xy, xz, yy, yz, zz,
            xx * x, xx * y, xx * z, xy * y, xy * z,
            xz * z, yy * y, yy * z, yz * z, zz * z]


def _mm(a, b):
    return jax.lax.dot_general(a, b, (((1,), (0,)), ((), ())),
                               preferred_element_type=jnp.float32)


def _knn_body(pc_ref, wt1_ref, bt1_ref, wt2_ref, bt2_ref,
              wt3_ref, bt3_ref, wt4_ref, bt4_ref,
              idx_ref, g1_ref, g2_ref, g3_ref, g4_ref, gco_ref):
    N = NPTS
    pcb = pc_ref[0]
    x3 = pcb[:, 0:3]

    sq_col = jnp.sum(x3 * x3, axis=1, keepdims=True)
    sq_row = jnp.sum(x3 * x3, axis=1)
    dot = jax.lax.dot_general(x3, x3, (((1,), (1,)), ((), ())),
                              preferred_element_type=jnp.float32)
    d2 = (sq_col + sq_row) - 2.0 * dot

    iota = jax.lax.broadcasted_iota(jnp.int32, (N, N), 1)
    d = d2
    for k in range(KNN):
        vmin = jnp.min(d, axis=1, keepdims=True)
        cand = jnp.where(d == vmin, iota, N)
        imin = jnp.min(cand, axis=1, keepdims=True)
        idx_ref[0, :, k:k + 1] = imin
        oh = jnp.where(iota == imin, 1.0, 0.0).astype(jnp.float32)
        gxyz = _mm(oh, x3) - x3
        gco_ref[0, :, k:k + 1] = gxyz[:, 0:1]
        gco_ref[1, :, k:k + 1] = gxyz[:, 1:2]
        gco_ref[2, :, k:k + 1] = gxyz[:, 2:3]
        d = jnp.where(iota == imin, jnp.float32(1e30), d)

    tms = _taylor_cols(gco_ref[0], gco_ref[1], gco_ref[2])

    for wt_ref, bt_ref, g_ref in ((wt1_ref, bt1_ref, g1_ref),
                                  (wt2_ref, bt2_ref, g2_ref),
                                  (wt3_ref, bt3_ref, g3_ref),
                                  (wt4_ref, bt4_ref, g4_ref)):
        for t in range(3):
            acc = tms[0] * wt_ref[t:t + 1, 0:1]
            for term in range(1, 20):
                acc = acc + tms[term] * wt_ref[t:t + 1, term:term + 1]
            g_ref[0, t] = acc + bt_ref[0:1, t:t + 1]


def _layer_body(feat_ref, idx_ref, g_ref, w_ref, bc_ref, out_ref):
    N = NPTS
    k = pl.program_id(1)
    feat = feat_ref[0]
    idxb = idx_ref[0]

    kio = jax.lax.broadcasted_iota(jnp.int32, (N, KNN), 1)
    kmask = kio == k
    colk = jnp.sum(jnp.where(kmask, idxb, 0), axis=1, keepdims=True)
    iota = jax.lax.broadcasted_iota(jnp.int32, (N, N), 1)
    oh = jnp.where(iota == colk, 1.0, 0.0).astype(jnp.float32)
    gf = _mm(oh, feat)

    gb = g_ref[0]
    zero = jnp.float32(0.0)
    g0 = jnp.sum(jnp.where(kmask, gb[0], zero), axis=1, keepdims=True)
    g1 = jnp.sum(jnp.where(kmask, gb[1], zero), axis=1, keepdims=True)
    g2 = jnp.sum(jnp.where(kmask, gb[2], zero), axis=1, keepdims=True)
    h = jnp.concatenate([gf * g0, gf * g1, gf * g2], axis=1)
    contrib = _mm(h, w_ref[0])

    @pl.when(k == 0)
    def _init():
        out_ref[0] = jnp.zeros_like(out_ref[0]) + bc_ref[...]

    out_ref[0] += contrib

    @pl.when(k == KNN - 1)
    def _relu():
        out_ref[0] = jnp.maximum(out_ref[0], 0.0)


def _top2_body(f1_ref, f2_ref, f3_ref, f4_ref, out_ref):
    N = NPTS
    cat = jnp.concatenate(
        [f1_ref[0], f2_ref[0], f3_ref[0], f4_ref[0]], axis=1)
    m1 = jnp.max(cat, axis=0, keepdims=True)
    riota = jax.lax.broadcasted_iota(jnp.int32, cat.shape, 0)
    ridx = jnp.min(jnp.where(cat == m1, riota, N), axis=0, keepdims=True)
    cat2 = jnp.where(riota == ridx, jnp.float32(-1e30), cat)
    m2 = jnp.max(cat2, axis=0, keepdims=True)
    out_ref[0, 0, :] = m1[0]
    out_ref[0, 1, :] = m2[0]


def _prep_layer(Wt, bt, Wc, bc):
    O, CT, K = Wc.shape
    C = CT // 3
    wflat = Wc.reshape(O, C, 3, K).transpose(3, 2, 1, 0).reshape(K, 3 * C, O)
    return (Wt, bt.reshape(1, 3), wflat, bc.reshape(1, O))


def _run_layer(feat, idx, g, wflat, bcr):
    B = feat.shape[0]
    C = feat.shape[2]
    O = bcr.shape[1]
    return pl.pallas_call(
        _layer_body,
        grid=(B, KNN),
        in_specs=[
            pl.BlockSpec((1, NPTS, C), lambda b, k: (b, 0, 0)),
            pl.BlockSpec((1, NPTS, KNN), lambda b, k: (b, 0, 0)),
            pl.BlockSpec((1, 3, NPTS, KNN), lambda b, k: (b, 0, 0, 0)),
            pl.BlockSpec((1, 3 * C, O), lambda b, k: (k, 0, 0)),
            pl.BlockSpec((1, O), lambda b, k: (0, 0)),
        ],
        out_specs=pl.BlockSpec((1, NPTS, O), lambda b, k: (b, 0, 0)),
        out_shape=jax.ShapeDtypeStruct((B, NPTS, O), jnp.float32),
    )(feat, idx, g, wflat, bcr)


def kernel(pc, Wt1, bt1, Wc1, bc1, Wt2, bt2, Wc2, bc2,
           Wt3, bt3, Wc3, bc3, Wt4, bt4, Wc4, bc4):
    B = pc.shape[0]
    layers = [_prep_layer(Wt1, bt1, Wc1, bc1),
              _prep_layer(Wt2, bt2, Wc2, bc2),
              _prep_layer(Wt3, bt3, Wc3, bc3),
              _prep_layer(Wt4, bt4, Wc4, bc4)]

    wt_args = []
    wt_specs = []
    for (wt, btr, _, _) in layers:
        wt_args += [wt, btr]
        wt_specs += [pl.BlockSpec((3, 20), lambda b: (0, 0)),
                     pl.BlockSpec((1, 3), lambda b: (0, 0))]

    gshape = jax.ShapeDtypeStruct((B, 3, NPTS, KNN), jnp.float32)
    idx, g1, g2, g3, g4 = pl.pallas_call(
        _knn_body,
        grid=(B,),
        in_specs=[pl.BlockSpec((1, NPTS, 6), lambda b: (b, 0, 0))] + wt_specs,
        out_specs=[
            pl.BlockSpec((1, NPTS, KNN), lambda b: (b, 0, 0)),
            pl.BlockSpec((1, 3, NPTS, KNN), lambda b: (b, 0, 0, 0)),
            pl.BlockSpec((1, 3, NPTS, KNN), lambda b: (b, 0, 0, 0)),
            pl.BlockSpec((1, 3, NPTS, KNN), lambda b: (b, 0, 0, 0)),
            pl.BlockSpec((1, 3, NPTS, KNN), lambda b: (b, 0, 0, 0)),
        ],
        out_shape=[
            jax.ShapeDtypeStruct((B, NPTS, KNN), jnp.int32),
            gshape, gshape, gshape, gshape,
        ],
        scratch_shapes=[pltpu.VMEM((3, NPTS, KNN), jnp.float32)],
    )(pc, *wt_args)

    feat = pc
    feats = []
    for g, (_, _, wflat, bcr) in zip((g1, g2, g3, g4), layers):
        feat = _run_layer(feat, idx, g, wflat, bcr)
        feats.append(feat)

    out = pl.pallas_call(
        _top2_body,
        grid=(B,),
        in_specs=[pl.BlockSpec((1, NPTS, f.shape[2]), lambda b: (b, 0, 0))
                  for f in feats],
        out_specs=pl.BlockSpec((1, 2, 480), lambda b: (b, 0, 0)),
        out_shape=jax.ShapeDtypeStruct((B, 2, 480), jnp.float32),
    )(*feats)
    return out.transpose(0, 2, 1).reshape(B, 960)


# speedup vs baseline: 1.2205x; 1.2205x over previous
import jax
import jax.numpy as jnp
from jax.experimental import pallas as pl
from jax.experimental.pallas import tpu as pltpu

KNN = 20
NPTS = 1024


def _taylor_cols(x, y, z):
    one = jnp.ones_like(x)
    xx = x * x
    xy = x * y
    xz = x * z
    yy = y * y
    yz = y * z
    zz = z * z
    return [one, x, y, z,
            xx, xy, xz, yy, yz, zz,
            xx * x, xx * y, xx * z, xy * y, xy * z,
            xz * z, yy * y, yy * z, yz * z, zz * z]


def _mm(a, b):
    return jax.lax.dot_general(a, b, (((1,), (0,)), ((), ())),
                               preferred_element_type=jnp.float32)


def _knn_body(pc_ref, wt1_ref, bt1_ref, wt2_ref, bt2_ref,
              wt3_ref, bt3_ref, wt4_ref, bt4_ref,
              idx_ref, g1_ref, g2_ref, g3_ref, g4_ref, gco_ref):
    N = NPTS
    pcb = pc_ref[0]
    x3 = pcb[:, 0:3]

    sq_col = jnp.sum(x3 * x3, axis=1, keepdims=True)
    sq_row = jnp.sum(x3 * x3, axis=1)
    dot = jax.lax.dot_general(x3, x3, (((1,), (1,)), ((), ())),
                              preferred_element_type=jnp.float32)
    d2 = (sq_col + sq_row) - 2.0 * dot

    iota = jax.lax.broadcasted_iota(jnp.int32, (N, N), 1)
    d = d2
    for k in range(KNN):
        vmin = jnp.min(d, axis=1, keepdims=True)
        cand = jnp.where(d == vmin, iota, N)
        imin = jnp.min(cand, axis=1, keepdims=True)
        idx_ref[0, :, k:k + 1] = imin
        oh = jnp.where(iota == imin, 1.0, 0.0).astype(jnp.float32)
        gxyz = _mm(oh, x3) - x3
        gco_ref[0, :, k:k + 1] = gxyz[:, 0:1]
        gco_ref[1, :, k:k + 1] = gxyz[:, 1:2]
        gco_ref[2, :, k:k + 1] = gxyz[:, 2:3]
        d = jnp.where(iota == imin, jnp.float32(1e30), d)

    tms = _taylor_cols(gco_ref[0], gco_ref[1], gco_ref[2])

    for wt_ref, bt_ref, g_ref in ((wt1_ref, bt1_ref, g1_ref),
                                  (wt2_ref, bt2_ref, g2_ref),
                                  (wt3_ref, bt3_ref, g3_ref),
                                  (wt4_ref, bt4_ref, g4_ref)):
        for t in range(3):
            acc = tms[0] * wt_ref[t:t + 1, 0:1]
            for term in range(1, 20):
                acc = acc + tms[term] * wt_ref[t:t + 1, term:term + 1]
            g_ref[0, t] = acc + bt_ref[0:1, t:t + 1]


def _layer_body(feat_ref, idx_ref, g_ref, w_ref, bc_ref, out_ref):
    N = NPTS
    feat = feat_ref[0]
    idxb = idx_ref[0]
    iota = jax.lax.broadcasted_iota(jnp.int32, (N, N), 1)

    acc = jnp.zeros((N, bc_ref.shape[1]), jnp.float32) + bc_ref[...]
    for k in range(KNN):
        oh = jnp.where(iota == idxb[:, k:k + 1], 1.0, 0.0).astype(jnp.float32)
        gf = _mm(oh, feat)
        g0 = g_ref[0, 0, :, k:k + 1]
        g1 = g_ref[0, 1, :, k:k + 1]
        g2 = g_ref[0, 2, :, k:k + 1]
        h = jnp.concatenate([gf * g0, gf * g1, gf * g2], axis=1)
        acc = acc + _mm(h, w_ref[k])
    out_ref[0] = jnp.maximum(acc, 0.0)


def _top2_body(f1_ref, f2_ref, f3_ref, f4_ref, out_ref):
    N = NPTS
    cat = jnp.concatenate(
        [f1_ref[0], f2_ref[0], f3_ref[0], f4_ref[0]], axis=1)
    m1 = jnp.max(cat, axis=0, keepdims=True)
    riota = jax.lax.broadcasted_iota(jnp.int32, cat.shape, 0)
    ridx = jnp.min(jnp.where(cat == m1, riota, N), axis=0, keepdims=True)
    cat2 = jnp.where(riota == ridx, jnp.float32(-1e30), cat)
    m2 = jnp.max(cat2, axis=0, keepdims=True)
    out_ref[0, 0, :] = m1[0]
    out_ref[0, 1, :] = m2[0]


def _prep_layer(Wt, bt, Wc, bc):
    O, CT, K = Wc.shape
    C = CT // 3
    wflat = Wc.reshape(O, C, 3, K).transpose(3, 2, 1, 0).reshape(K, 3 * C, O)
    return (Wt, bt.reshape(1, 3), wflat, bc.reshape(1, O))


def _run_layer(feat, idx, g, wflat, bcr):
    B = feat.shape[0]
    C = feat.shape[2]
    O = bcr.shape[1]
    return pl.pallas_call(
        _layer_body,
        grid=(B,),
        in_specs=[
            pl.BlockSpec((1, NPTS, C), lambda b: (b, 0, 0)),
            pl.BlockSpec((1, NPTS, KNN), lambda b: (b, 0, 0)),
            pl.BlockSpec((1, 3, NPTS, KNN), lambda b: (b, 0, 0, 0)),
            pl.BlockSpec((KNN, 3 * C, O), lambda b: (0, 0, 0)),
            pl.BlockSpec((1, O), lambda b: (0, 0)),
        ],
        out_specs=pl.BlockSpec((1, NPTS, O), lambda b: (b, 0, 0)),
        out_shape=jax.ShapeDtypeStruct((B, NPTS, O), jnp.float32),
    )(feat, idx, g, wflat, bcr)


def kernel(pc, Wt1, bt1, Wc1, bc1, Wt2, bt2, Wc2, bc2,
           Wt3, bt3, Wc3, bc3, Wt4, bt4, Wc4, bc4):
    B = pc.shape[0]
    layers = [_prep_layer(Wt1, bt1, Wc1, bc1),
              _prep_layer(Wt2, bt2, Wc2, bc2),
              _prep_layer(Wt3, bt3, Wc3, bc3),
              _prep_layer(Wt4, bt4, Wc4, bc4)]

    wt_args = []
    wt_specs = []
    for (wt, btr, _, _) in layers:
        wt_args += [wt, btr]
        wt_specs += [pl.BlockSpec((3, 20), lambda b: (0, 0)),
                     pl.BlockSpec((1, 3), lambda b: (0, 0))]

    gshape = jax.ShapeDtypeStruct((B, 3, NPTS, KNN), jnp.float32)
    idx, g1, g2, g3, g4 = pl.pallas_call(
        _knn_body,
        grid=(B,),
        in_specs=[pl.BlockSpec((1, NPTS, 6), lambda b: (b, 0, 0))] + wt_specs,
        out_specs=[
            pl.BlockSpec((1, NPTS, KNN), lambda b: (b, 0, 0)),
            pl.BlockSpec((1, 3, NPTS, KNN), lambda b: (b, 0, 0, 0)),
            pl.BlockSpec((1, 3, NPTS, KNN), lambda b: (b, 0, 0, 0)),
            pl.BlockSpec((1, 3, NPTS, KNN), lambda b: (b, 0, 0, 0)),
            pl.BlockSpec((1, 3, NPTS, KNN), lambda b: (b, 0, 0, 0)),
        ],
        out_shape=[
            jax.ShapeDtypeStruct((B, NPTS, KNN), jnp.int32),
            gshape, gshape, gshape, gshape,
        ],
        scratch_shapes=[pltpu.VMEM((3, NPTS, KNN), jnp.float32)],
    )(pc, *wt_args)

    feat = pc
    feats = []
    for g, (_, _, wflat, bcr) in zip((g1, g2, g3, g4), layers):
        feat = _run_layer(feat, idx, g, wflat, bcr)
        feats.append(feat)

    out = pl.pallas_call(
        _top2_body,
        grid=(B,),
        in_specs=[pl.BlockSpec((1, NPTS, f.shape[2]), lambda b: (b, 0, 0))
                  for f in feats],
        out_specs=pl.BlockSpec((1, 2, 480), lambda b: (b, 0, 0)),
        out_shape=jax.ShapeDtypeStruct((B, 2, 480), jnp.float32),
    )(*feats)
    return out.transpose(0, 2, 1).reshape(B, 960)


# speedup vs baseline: 1.3352x; 1.0940x over previous
import jax
import jax.numpy as jnp
from jax.experimental import pallas as pl
from jax.experimental.pallas import tpu as pltpu

KNN = 20
NPTS = 1024


def _taylor_cols(x, y, z):
    one = jnp.ones_like(x)
    xx = x * x
    xy = x * y
    xz = x * z
    yy = y * y
    yz = y * z
    zz = z * z
    return [one, x, y, z,
            xx, xy, xz, yy, yz, zz,
            xx * x, xx * y, xx * z, xy * y, xy * z,
            xz * z, yy * y, yy * z, yz * z, zz * z]


def _mm(a, b):
    return jax.lax.dot_general(a, b, (((1,), (0,)), ((), ())),
                               preferred_element_type=jnp.float32)


def _knn_body(pc_ref, wt1_ref, bt1_ref, wt2_ref, bt2_ref,
              wt3_ref, bt3_ref, wt4_ref, bt4_ref,
              idx_ref, g1_ref, g2_ref, g3_ref, g4_ref, gco_ref):
    N = NPTS
    pcb = pc_ref[0]
    x3 = pcb[:, 0:3]

    sq_col = jnp.sum(x3 * x3, axis=1, keepdims=True)
    sq_row = jnp.sum(x3 * x3, axis=1)
    dot = jax.lax.dot_general(x3, x3, (((1,), (1,)), ((), ())),
                              preferred_element_type=jnp.float32)
    d2 = (sq_col + sq_row) - 2.0 * dot

    iota = jax.lax.broadcasted_iota(jnp.int32, (N, N), 1)
    d = d2
    for k in range(KNN):
        vmin = jnp.min(d, axis=1, keepdims=True)
        cand = jnp.where(d == vmin, iota, N)
        imin = jnp.min(cand, axis=1, keepdims=True)
        idx_ref[0, :, k:k + 1] = imin
        oh = jnp.where(iota == imin, 1.0, 0.0).astype(jnp.float32)
        gxyz = _mm(oh, x3) - x3
        gco_ref[0, :, k:k + 1] = gxyz[:, 0:1]
        gco_ref[1, :, k:k + 1] = gxyz[:, 1:2]
        gco_ref[2, :, k:k + 1] = gxyz[:, 2:3]
        d = jnp.where(iota == imin, jnp.float32(1e30), d)

    tms = _taylor_cols(gco_ref[0], gco_ref[1], gco_ref[2])

    for wt_ref, bt_ref, g_ref in ((wt1_ref, bt1_ref, g1_ref),
                                  (wt2_ref, bt2_ref, g2_ref),
                                  (wt3_ref, bt3_ref, g3_ref),
                                  (wt4_ref, bt4_ref, g4_ref)):
        for t in range(3):
            acc = tms[0] * wt_ref[t:t + 1, 0:1]
            for term in range(1, 20):
                acc = acc + tms[term] * wt_ref[t:t + 1, term:term + 1]
            g_ref[0, t] = acc + bt_ref[0:1, t:t + 1]


def _layer_body(feat_ref, idx_ref, g_ref, w_ref, bc_ref, out_ref):
    N = NPTS
    feat = feat_ref[0]
    idxb = idx_ref[0]
    iota = jax.lax.broadcasted_iota(jnp.int32, (N, N), 1)

    acc = jnp.zeros((N, bc_ref.shape[1]), jnp.float32) + bc_ref[...]
    for k in range(KNN):
        oh = jnp.where(iota == idxb[:, k:k + 1], 1.0, 0.0).astype(jnp.float32)
        gf = _mm(oh, feat)
        g0 = g_ref[0, 0, :, k:k + 1]
        g1 = g_ref[0, 1, :, k:k + 1]
        g2 = g_ref[0, 2, :, k:k + 1]
        acc = acc + (_mm(gf * g0, w_ref[k, 0]) + _mm(gf * g1, w_ref[k, 1])
                     + _mm(gf * g2, w_ref[k, 2]))
    out_ref[0] = jnp.maximum(acc, 0.0)


def _top2_body(f1_ref, f2_ref, f3_ref, f4_ref, out_ref):
    N = NPTS
    cat = jnp.concatenate(
        [f1_ref[0], f2_ref[0], f3_ref[0], f4_ref[0]], axis=1)
    m1 = jnp.max(cat, axis=0, keepdims=True)
    riota = jax.lax.broadcasted_iota(jnp.int32, cat.shape, 0)
    ridx = jnp.min(jnp.where(cat == m1, riota, N), axis=0, keepdims=True)
    cat2 = jnp.where(riota == ridx, jnp.float32(-1e30), cat)
    m2 = jnp.max(cat2, axis=0, keepdims=True)
    out_ref[0, 0, :] = m1[0]
    out_ref[0, 1, :] = m2[0]


def _prep_layer(Wt, bt, Wc, bc):
    O, CT, K = Wc.shape
    C = CT // 3
    wflat = Wc.reshape(O, C, 3, K).transpose(3, 2, 1, 0)
    return (Wt, bt.reshape(1, 3), wflat, bc.reshape(1, O))


def _run_layer(feat, idx, g, wflat, bcr):
    B = feat.shape[0]
    C = feat.shape[2]
    O = bcr.shape[1]
    return pl.pallas_call(
        _layer_body,
        grid=(B,),
        in_specs=[
            pl.BlockSpec((1, NPTS, C), lambda b: (b, 0, 0)),
            pl.BlockSpec((1, NPTS, KNN), lambda b: (b, 0, 0)),
            pl.BlockSpec((1, 3, NPTS, KNN), lambda b: (b, 0, 0, 0)),
            pl.BlockSpec((KNN, 3, C, O), lambda b: (0, 0, 0, 0)),
            pl.BlockSpec((1, O), lambda b: (0, 0)),
        ],
        out_specs=pl.BlockSpec((1, NPTS, O), lambda b: (b, 0, 0)),
        out_shape=jax.ShapeDtypeStruct((B, NPTS, O), jnp.float32),
    )(feat, idx, g, wflat, bcr)


def kernel(pc, Wt1, bt1, Wc1, bc1, Wt2, bt2, Wc2, bc2,
           Wt3, bt3, Wc3, bc3, Wt4, bt4, Wc4, bc4):
    B = pc.shape[0]
    layers = [_prep_layer(Wt1, bt1, Wc1, bc1),
              _prep_layer(Wt2, bt2, Wc2, bc2),
              _prep_layer(Wt3, bt3, Wc3, bc3),
              _prep_layer(Wt4, bt4, Wc4, bc4)]

    wt_args = []
    wt_specs = []
    for (wt, btr, _, _) in layers:
        wt_args += [wt, btr]
        wt_specs += [pl.BlockSpec((3, 20), lambda b: (0, 0)),
                     pl.BlockSpec((1, 3), lambda b: (0, 0))]

    gshape = jax.ShapeDtypeStruct((B, 3, NPTS, KNN), jnp.float32)
    idx, g1, g2, g3, g4 = pl.pallas_call(
        _knn_body,
        grid=(B,),
        in_specs=[pl.BlockSpec((1, NPTS, 6), lambda b: (b, 0, 0))] + wt_specs,
        out_specs=[
            pl.BlockSpec((1, NPTS, KNN), lambda b: (b, 0, 0)),
            pl.BlockSpec((1, 3, NPTS, KNN), lambda b: (b, 0, 0, 0)),
            pl.BlockSpec((1, 3, NPTS, KNN), lambda b: (b, 0, 0, 0)),
            pl.BlockSpec((1, 3, NPTS, KNN), lambda b: (b, 0, 0, 0)),
            pl.BlockSpec((1, 3, NPTS, KNN), lambda b: (b, 0, 0, 0)),
        ],
        out_shape=[
            jax.ShapeDtypeStruct((B, NPTS, KNN), jnp.int32),
            gshape, gshape, gshape, gshape,
        ],
        scratch_shapes=[pltpu.VMEM((3, NPTS, KNN), jnp.float32)],
    )(pc, *wt_args)

    feat = pc
    feats = []
    for g, (_, _, wflat, bcr) in zip((g1, g2, g3, g4), layers):
        feat = _run_layer(feat, idx, g, wflat, bcr)
        feats.append(feat)

    out = pl.pallas_call(
        _top2_body,
        grid=(B,),
        in_specs=[pl.BlockSpec((1, NPTS, f.shape[2]), lambda b: (b, 0, 0))
                  for f in feats],
        out_specs=pl.BlockSpec((1, 2, 480), lambda b: (b, 0, 0)),
        out_shape=jax.ShapeDtypeStruct((B, 2, 480), jnp.float32),
    )(*feats)
    return out.transpose(0, 2, 1).reshape(B, 960)
